# Initial kernel scaffold; baseline (speedup 1.0000x reference)
import jax
import jax.numpy as jnp
from jax.experimental import pallas as pl


def kernel(x, edge_index, W_in, b_in, W1, b1, W2, b2, W3, b3):
    raise NotImplementedError("write your pallas kernel here")


# speedup vs baseline: 1.7828x; 1.7828x over previous
import functools

import jax
import jax.numpy as jnp
from jax.experimental import pallas as pl

N_NODES = 10000
D = 128
NEG_SLOPE = 0.01
ROW_BLK = 1000


def _stage_a_body(x_ref, deg_ref, Win_ref, bin_ref, W1_ref, o_ref):
    t = jnp.dot(x_ref[...], Win_ref[...],
                preferred_element_type=jnp.float32) + bin_ref[...]
    u = jnp.dot(t, W1_ref[...], preferred_element_type=jnp.float32)
    dis = jax.lax.rsqrt(deg_ref[...])
    o_ref[...] = u * dis


def _stage_mid_body(p0_ref, p1_ref, deg_ref, b_ref, W_ref, o_ref):
    dis = jax.lax.rsqrt(deg_ref[...])
    g = (p0_ref[...] + p1_ref[...]) * dis + b_ref[...]
    g = jnp.where(g >= 0, g, NEG_SLOPE * g)
    u = jnp.dot(g, W_ref[...], preferred_element_type=jnp.float32)
    o_ref[...] = u * dis


def _stage_c_body(p0_ref, p1_ref, deg_ref, b_ref, o_ref):
    dis = jax.lax.rsqrt(deg_ref[...])
    o_ref[...] = (p0_ref[...] + p1_ref[...]) * dis + b_ref[...]


_row_spec = pl.BlockSpec((ROW_BLK, D), lambda i: (i, 0))
_deg_spec = pl.BlockSpec((ROW_BLK, 1), lambda i: (i, 0))
_w_spec = pl.BlockSpec((D, D), lambda i: (0, 0))
_b_spec = pl.BlockSpec((1, D), lambda i: (0, 0))
_out_sds = jax.ShapeDtypeStruct((N_NODES, D), jnp.float32)
_grid = (N_NODES // ROW_BLK,)


def _stage_a(x, deg2, W_in, b_in, W1):
    return pl.pallas_call(
        _stage_a_body, grid=_grid,
        in_specs=[_row_spec, _deg_spec, _w_spec, _b_spec, _w_spec],
        out_specs=_row_spec, out_shape=_out_sds,
    )(x, deg2, W_in, b_in[None, :], W1)


def _stage_mid(p0, p1, deg2, b, W):
    return pl.pallas_call(
        _stage_mid_body, grid=_grid,
        in_specs=[_row_spec, _row_spec, _deg_spec, _b_spec, _w_spec],
        out_specs=_row_spec, out_shape=_out_sds,
    )(p0, p1, deg2, b[None, :], W)


def _stage_c(p0, p1, deg2, b):
    return pl.pallas_call(
        _stage_c_body, grid=_grid,
        in_specs=[_row_spec, _row_spec, _deg_spec, _b_spec],
        out_specs=_row_spec, out_shape=_out_sds,
    )(p0, p1, deg2, b[None, :])


def _seg_sum(h, src, dst):
    return jax.ops.segment_sum(h[src], dst, num_segments=N_NODES)


def kernel(x, edge_index, W_in, b_in, W1, b1, W2, b2, W3, b3):
    src = edge_index[0].astype(jnp.int32)
    dst = edge_index[1].astype(jnp.int32)
    loop = jnp.arange(N_NODES, dtype=jnp.int32)
    src = jnp.concatenate([src, loop])
    dst = jnp.concatenate([dst, loop])

    deg = jax.ops.segment_sum(jnp.ones_like(dst, dtype=jnp.float32), dst,
                              num_segments=N_NODES)
    deg2 = deg[:, None]

    hs = _stage_a(x, deg2, W_in, b_in, W1)
    p = _seg_sum(hs, src, dst)
    hs = _stage_mid(p, jnp.zeros_like(p), deg2, b1, W2)
    p = _seg_sum(hs, src, dst)
    hs = _stage_mid(p, jnp.zeros_like(p), deg2, b2, W3)
    p = _seg_sum(hs, src, dst)
    return _stage_c(p, jnp.zeros_like(p), deg2, b3)


# speedup vs baseline: 20.2136x; 11.3382x over previous
import functools

import jax
import jax.numpy as jnp
from jax import lax
from jax.experimental import pallas as pl
from jax.experimental.pallas import tpu as pltpu
from jax.experimental.pallas import tpu_sc as plsc

N_NODES = 10000
D = 128
DH = D // 2
NEG_SLOPE = 0.01
ROW_BLK = 1000

NC, NS = 2, 16
CHUNK = 128
GB = 8
NBLK = 21
CPW = NBLK * GB
NBLK_TOT = NBLK + 2
NBUF = 4
LEAD = 2
SDEPTH = NBUF - LEAD
E_REAL = 320000 + N_NODES
E_SCAT = NS * CPW * CHUNK
E_PAD = NS * NBLK_TOT * GB * CHUNK
N_PAD = 10240
JUNK = N_NODES
ZROWS = N_PAD // NS
TROWS = N_NODES // NS

_sc_mesh = plsc.VectorSubcoreMesh(
    core_axis_name="c", subcore_axis_name="s", num_cores=NC, num_subcores=NS)


@functools.partial(
    pl.kernel,
    out_type=jax.ShapeDtypeStruct((NC, N_PAD, DH), jnp.float32),
    mesh=_sc_mesh,
    compiler_params=pltpu.CompilerParams(use_tc_tiling_on_sc=False),
    scratch_types=[
        pltpu.VMEM((3, GB, CHUNK), jnp.int32),
        pltpu.VMEM((3, GB, CHUNK), jnp.int32),
        [pltpu.VMEM((CHUNK, DH), jnp.float32)] * NBUF,
        pltpu.VMEM_SHARED((N_NODES, DH), jnp.float32),
        pltpu.VMEM_SHARED((N_PAD, DH), jnp.float32),
        [pltpu.SemaphoreType.DMA] * NBUF,
        [pltpu.SemaphoreType.DMA] * NBUF,
        [pltpu.SemaphoreType.DMA] * 3,
    ],
)
def _sc_segsum(hs_hbm, src_hbm, dst_hbm, out_hbm,
               src_i, dst_i, bufs, table, acc, gsems, ssems, isems):
    cid = lax.axis_index("c")
    sid = lax.axis_index("s")

    def _zrow(j, _):
        for k in range(DH // 16):
            bufs[0][j, pl.ds(k * 16, 16)] = jnp.zeros((16,), jnp.float32)
        return 0
    lax.fori_loop(0, CHUNK, _zrow, 0)

    def _zcopy(k, _):
        pltpu.sync_copy(bufs[0],
                        acc.at[pl.ds(sid * ZROWS + k * CHUNK, CHUNK)])
        return 0
    lax.fori_loop(0, ZROWS // CHUNK, _zcopy, 0)

    pltpu.sync_copy(hs_hbm.at[cid, pl.ds(sid * TROWS, TROWS)],
                    table.at[pl.ds(sid * TROWS, TROWS)])

    pltpu.sync_copy(src_hbm.at[sid, 0], src_i.at[0])
    pltpu.sync_copy(dst_hbm.at[sid, 0], dst_i.at[0])

    def _idx_load(j, q):
        pltpu.async_copy(src_hbm.at[sid, j], src_i.at[q], isems[q])
        pltpu.async_copy(dst_hbm.at[sid, j], dst_i.at[q], isems[q])

    def _idx_wait(q):
        pltpu.make_async_copy(src_hbm.at[sid, 0], src_i.at[q],
                              isems[q]).wait()
        pltpu.make_async_copy(dst_hbm.at[sid, 0], dst_i.at[q],
                              isems[q]).wait()

    _idx_load(1, 1)

    plsc.subcore_barrier()

    def _gather(q, r, b):
        pltpu.async_copy(table.at[src_i.at[q, r]], bufs[b], gsems[b])

    def _gwait(b):
        pltpu.make_async_copy(table.at[src_i.at[0, 0]], bufs[b],
                              gsems[b]).wait()

    def _scatter(q, r, b):
        pltpu.async_copy(bufs[b], acc.at[dst_i.at[q, r]], ssems[b],
                         add=True)

    def _swait(b):
        pltpu.make_async_copy(bufs[0], acc.at[dst_i.at[0, 0]],
                              ssems[b]).wait()

    def _block(j, q, first=False):
        qn = (q + 1) % 3
        for k in range(GB):
            b = k % NBUF
            bn = (k + LEAD) % NBUF
            if k == 2:
                _idx_load(j + 2, (q + 2) % 3)
            if k == GB - LEAD:
                _idx_wait(qn)
            if not (first and k < SDEPTH):
                _swait(bn)
            if k + LEAD < GB:
                _gather(q, k + LEAD, bn)
            else:
                _gather(qn, k + LEAD - GB, bn)
            _gwait(b)
            _scatter(q, k, b)

    for b in range(LEAD):
        _gather(0, b, b)

    _block(0, 0, first=True)

    def _step3(i, _):
        j0 = 3 * i + 1
        _block(j0, 1)
        _block(j0 + 1, 2)
        _block(j0 + 2, 0)
        return 0
    lax.fori_loop(0, (NBLK - 3) // 3, _step3, 0)
    _block(NBLK - 2, (NBLK - 2) % 3)
    _block(NBLK - 1, (NBLK - 1) % 3)

    for k in range(SDEPTH):
        _swait((GB - SDEPTH + k) % NBUF)
    for k in range(LEAD):
        _gwait(k % NBUF)
    _idx_wait((NBLK + 1) % 3)

    plsc.subcore_barrier()

    pltpu.sync_copy(acc.at[pl.ds(sid * ZROWS, ZROWS)],
                    out_hbm.at[cid, pl.ds(sid * ZROWS, ZROWS)])


@functools.partial(
    pl.kernel,
    out_type=jax.ShapeDtypeStruct((NC, N_PAD, 16), jnp.float32),
    mesh=_sc_mesh,
    scratch_types=[
        pltpu.VMEM((NBLK_TOT, GB, CHUNK), jnp.int32),
        pltpu.VMEM((CHUNK, 16), jnp.float32),
        pltpu.VMEM_SHARED((N_PAD, 16), jnp.float32),
    ],
)
def _sc_degree(dst_hbm, out_hbm, dst_all, ones_v, acc):
    cid = lax.axis_index("c")
    sid = lax.axis_index("s")

    def _fill(val):
        def _f(j, _):
            ones_v[j, pl.ds(0, 16)] = jnp.full((16,), val, jnp.float32)
            return 0
        lax.fori_loop(0, CHUNK, _f, 0)

    _fill(0.0)

    def _zcopy(k, _):
        pltpu.sync_copy(ones_v,
                        acc.at[pl.ds(sid * ZROWS + k * CHUNK, CHUNK)])
        return 0
    lax.fori_loop(0, ZROWS // CHUNK, _zcopy, 0)

    pltpu.sync_copy(dst_hbm.at[sid], dst_all)
    _fill(1.0)

    plsc.subcore_barrier()

    def _step(j, _):
        for r in range(GB):
            pltpu.sync_copy(ones_v, acc.at[dst_all.at[j, r]], add=True)
        return 0
    lax.fori_loop(0, NBLK, _step, 0)

    plsc.subcore_barrier()

    pltpu.sync_copy(acc.at[pl.ds(sid * ZROWS, ZROWS)],
                    out_hbm.at[cid, pl.ds(sid * ZROWS, ZROWS)])


def _stage_a_body(x_ref, deg_ref, Win_ref, bin_ref, W1h_ref, o_ref):
    t = jnp.dot(x_ref[...], Win_ref[...],
                preferred_element_type=jnp.float32) + bin_ref[...]
    u = jnp.dot(t, W1h_ref[0], preferred_element_type=jnp.float32)
    dis = jax.lax.rsqrt(deg_ref[...])
    o_ref[0] = u * dis


def _stage_mid_body(p_ref, deg_ref, b_ref, Wh_ref, o_ref):
    dis = jax.lax.rsqrt(deg_ref[...])
    g = jnp.concatenate([p_ref[0], p_ref[1]], axis=1) * dis + b_ref[...]
    g = jnp.where(g >= 0, g, NEG_SLOPE * g)
    u = jnp.dot(g, Wh_ref[0], preferred_element_type=jnp.float32)
    o_ref[0] = u * dis


def _stage_c_body(p_ref, deg_ref, b_ref, o_ref):
    dis = jax.lax.rsqrt(deg_ref[...])
    o_ref[...] = jnp.concatenate([p_ref[0], p_ref[1]], axis=1) * dis + b_ref[...]


_x_spec = pl.BlockSpec((ROW_BLK, D), lambda i, j: (i, 0))
_p_spec = pl.BlockSpec((NC, ROW_BLK, DH), lambda i, j: (0, i, 0))
_deg_spec = pl.BlockSpec((ROW_BLK, 1), lambda i, j: (i, 0))
_wfull_spec = pl.BlockSpec((D, D), lambda i, j: (0, 0))
_whalf_spec = pl.BlockSpec((1, D, DH), lambda i, j: (j, 0, 0))
_bfull_spec = pl.BlockSpec((1, D), lambda i, j: (0, 0))
_osplit_spec = pl.BlockSpec((1, ROW_BLK, DH), lambda i, j: (j, i, 0))
_osplit_sds = jax.ShapeDtypeStruct((NC, N_NODES, DH), jnp.float32)
_grid = (N_NODES // ROW_BLK, NC)


def _split_w(W):
    return W.reshape(D, NC, DH).transpose(1, 0, 2)


def _stage_a(x, deg2, W_in, b_in, W1):
    return pl.pallas_call(
        _stage_a_body, grid=_grid,
        in_specs=[_x_spec, _deg_spec, _wfull_spec, _bfull_spec, _whalf_spec],
        out_specs=_osplit_spec, out_shape=_osplit_sds,
    )(x, deg2, W_in, b_in[None, :], _split_w(W1))


def _stage_mid(p, deg2, b, W):
    return pl.pallas_call(
        _stage_mid_body, grid=_grid,
        in_specs=[_p_spec, _deg_spec, _bfull_spec, _whalf_spec],
        out_specs=_osplit_spec, out_shape=_osplit_sds,
    )(p, deg2, b[None, :], _split_w(W))


def _stage_c(p, deg2, b):
    return pl.pallas_call(
        _stage_c_body, grid=(N_NODES // ROW_BLK,),
        in_specs=[pl.BlockSpec((NC, ROW_BLK, DH), lambda i: (0, i, 0)),
                  pl.BlockSpec((ROW_BLK, 1), lambda i: (i, 0)),
                  pl.BlockSpec((1, D), lambda i: (0, 0))],
        out_specs=pl.BlockSpec((ROW_BLK, D), lambda i: (i, 0)),
        out_shape=jax.ShapeDtypeStruct((N_NODES, D), jnp.float32),
    )(p, deg2, b[None, :])


def kernel(x, edge_index, W_in, b_in, W1, b1, W2, b2, W3, b3):
    src = edge_index[0].astype(jnp.int32)
    dst = edge_index[1].astype(jnp.int32)
    loop = jnp.arange(N_NODES, dtype=jnp.int32)

    src_p = jnp.concatenate(
        [src, loop, jnp.zeros((E_SCAT - E_REAL,), jnp.int32)])
    dst_p = jnp.concatenate(
        [dst, loop, jnp.full((E_SCAT - E_REAL,), JUNK, jnp.int32)])
    src_w = jnp.pad(src_p.reshape(NS, NBLK, GB, CHUNK),
                    ((0, 0), (0, NBLK_TOT - NBLK), (0, 0), (0, 0)))
    dst_w = jnp.pad(dst_p.reshape(NS, NBLK, GB, CHUNK),
                    ((0, 0), (0, NBLK_TOT - NBLK), (0, 0), (0, 0)),
                    constant_values=JUNK)

    pdeg = _sc_degree(dst_w)
    deg2 = pdeg[0, :N_NODES, 0:1]

    hs = _stage_a(x, deg2, W_in, b_in, W1)
    p = _sc_segsum(hs, src_w, dst_w)
    hs = _stage_mid(p, deg2, b1, W2)
    p = _sc_segsum(hs, src_w, dst_w)
    hs = _stage_mid(p, deg2, b2, W3)
    p = _sc_segsum(hs, src_w, dst_w)
    return _stage_c(p, deg2, b3)


# speedup vs baseline: 20.3184x; 1.0052x over previous
import functools

import jax
import jax.numpy as jnp
from jax import lax
from jax.experimental import pallas as pl
from jax.experimental.pallas import tpu as pltpu
from jax.experimental.pallas import tpu_sc as plsc

N_NODES = 10000
D = 128
DH = D // 2
NEG_SLOPE = 0.01
ROW_BLK = 1000

NC, NS = 2, 16
CHUNK = 128
GB = 8
NBLK = 21
CPW = NBLK * GB
NBLK_TOT = NBLK + 2
NBUF = 4
LEAD = 1
SDEPTH = NBUF - LEAD
E_REAL = 320000 + N_NODES
E_SCAT = NS * CPW * CHUNK
E_PAD = NS * NBLK_TOT * GB * CHUNK
N_PAD = 10240
JUNK = N_NODES
ZROWS = N_PAD // NS
TROWS = N_NODES // NS

_sc_mesh = plsc.VectorSubcoreMesh(
    core_axis_name="c", subcore_axis_name="s", num_cores=NC, num_subcores=NS)


@functools.partial(
    pl.kernel,
    out_type=jax.ShapeDtypeStruct((NC, N_PAD, DH), jnp.float32),
    mesh=_sc_mesh,
    compiler_params=pltpu.CompilerParams(use_tc_tiling_on_sc=False),
    scratch_types=[
        pltpu.VMEM((3, GB, CHUNK), jnp.int32),
        pltpu.VMEM((3, GB, CHUNK), jnp.int32),
        [pltpu.VMEM((CHUNK, DH), jnp.float32)] * NBUF,
        pltpu.VMEM_SHARED((N_NODES, DH), jnp.float32),
        pltpu.VMEM_SHARED((N_PAD, DH), jnp.float32),
        [pltpu.SemaphoreType.DMA] * NBUF,
        [pltpu.SemaphoreType.DMA] * NBUF,
        [pltpu.SemaphoreType.DMA] * 3,
    ],
)
def _sc_segsum(hs_hbm, src_hbm, dst_hbm, out_hbm,
               src_i, dst_i, bufs, table, acc, gsems, ssems, isems):
    cid = lax.axis_index("c")
    sid = lax.axis_index("s")

    def _zrow(j, _):
        for k in range(DH // 16):
            bufs[0][j, pl.ds(k * 16, 16)] = jnp.zeros((16,), jnp.float32)
        return 0
    lax.fori_loop(0, CHUNK, _zrow, 0)

    def _zcopy(k, _):
        pltpu.sync_copy(bufs[0],
                        acc.at[pl.ds(sid * ZROWS + k * CHUNK, CHUNK)])
        return 0
    lax.fori_loop(0, ZROWS // CHUNK, _zcopy, 0)

    pltpu.sync_copy(hs_hbm.at[cid, pl.ds(sid * TROWS, TROWS)],
                    table.at[pl.ds(sid * TROWS, TROWS)])

    pltpu.sync_copy(src_hbm.at[sid, 0], src_i.at[0])
    pltpu.sync_copy(dst_hbm.at[sid, 0], dst_i.at[0])

    def _idx_load(j, q):
        pltpu.async_copy(src_hbm.at[sid, j], src_i.at[q], isems[q])
        pltpu.async_copy(dst_hbm.at[sid, j], dst_i.at[q], isems[q])

    def _idx_wait(q):
        pltpu.make_async_copy(src_hbm.at[sid, 0], src_i.at[q],
                              isems[q]).wait()
        pltpu.make_async_copy(dst_hbm.at[sid, 0], dst_i.at[q],
                              isems[q]).wait()

    _idx_load(1, 1)

    plsc.subcore_barrier()

    def _gather(q, r, b):
        pltpu.async_copy(table.at[src_i.at[q, r]], bufs[b], gsems[b])

    def _gwait(b):
        pltpu.make_async_copy(table.at[src_i.at[0, 0]], bufs[b],
                              gsems[b]).wait()

    def _scatter(q, r, b):
        pltpu.async_copy(bufs[b], acc.at[dst_i.at[q, r]], ssems[b],
                         add=True)

    def _swait(b):
        pltpu.make_async_copy(bufs[0], acc.at[dst_i.at[0, 0]],
                              ssems[b]).wait()

    def _block(j, q, first=False):
        qn = (q + 1) % 3
        for k in range(GB):
            b = k % NBUF
            bn = (k + LEAD) % NBUF
            if k == 2:
                _idx_load(j + 2, (q + 2) % 3)
            if k == GB - LEAD:
                _idx_wait(qn)
            if not (first and k < SDEPTH):
                _swait(bn)
            if k + LEAD < GB:
                _gather(q, k + LEAD, bn)
            else:
                _gather(qn, k + LEAD - GB, bn)
            _gwait(b)
            _scatter(q, k, b)

    for b in range(LEAD):
        _gather(0, b, b)

    _block(0, 0, first=True)

    def _step3(i, _):
        j0 = 3 * i + 1
        _block(j0, 1)
        _block(j0 + 1, 2)
        _block(j0 + 2, 0)
        return 0
    lax.fori_loop(0, (NBLK - 3) // 3, _step3, 0)
    _block(NBLK - 2, (NBLK - 2) % 3)
    _block(NBLK - 1, (NBLK - 1) % 3)

    for k in range(SDEPTH):
        _swait((GB - SDEPTH + k) % NBUF)
    for k in range(LEAD):
        _gwait(k % NBUF)
    _idx_wait((NBLK + 1) % 3)

    plsc.subcore_barrier()

    pltpu.sync_copy(acc.at[pl.ds(sid * ZROWS, ZROWS)],
                    out_hbm.at[cid, pl.ds(sid * ZROWS, ZROWS)])


@functools.partial(
    pl.kernel,
    out_type=jax.ShapeDtypeStruct((NC, N_PAD, 16), jnp.float32),
    mesh=_sc_mesh,
    scratch_types=[
        pltpu.VMEM((NBLK_TOT, GB, CHUNK), jnp.int32),
        pltpu.VMEM((CHUNK, 16), jnp.float32),
        pltpu.VMEM_SHARED((N_PAD, 16), jnp.float32),
    ],
)
def _sc_degree(dst_hbm, out_hbm, dst_all, ones_v, acc):
    cid = lax.axis_index("c")
    sid = lax.axis_index("s")

    def _fill(val):
        def _f(j, _):
            ones_v[j, pl.ds(0, 16)] = jnp.full((16,), val, jnp.float32)
            return 0
        lax.fori_loop(0, CHUNK, _f, 0)

    _fill(0.0)

    def _zcopy(k, _):
        pltpu.sync_copy(ones_v,
                        acc.at[pl.ds(sid * ZROWS + k * CHUNK, CHUNK)])
        return 0
    lax.fori_loop(0, ZROWS // CHUNK, _zcopy, 0)

    pltpu.sync_copy(dst_hbm.at[sid], dst_all)
    _fill(1.0)

    plsc.subcore_barrier()

    def _step(j, _):
        for r in range(GB):
            pltpu.sync_copy(ones_v, acc.at[dst_all.at[j, r]], add=True)
        return 0
    lax.fori_loop(0, NBLK, _step, 0)

    plsc.subcore_barrier()

    pltpu.sync_copy(acc.at[pl.ds(sid * ZROWS, ZROWS)],
                    out_hbm.at[cid, pl.ds(sid * ZROWS, ZROWS)])


def _stage_a_body(x_ref, deg_ref, Win_ref, bin_ref, W1h_ref, o_ref):
    t = jnp.dot(x_ref[...], Win_ref[...],
                preferred_element_type=jnp.float32) + bin_ref[...]
    u = jnp.dot(t, W1h_ref[0], preferred_element_type=jnp.float32)
    dis = jax.lax.rsqrt(deg_ref[...])
    o_ref[0] = u * dis


def _stage_mid_body(p_ref, deg_ref, b_ref, Wh_ref, o_ref):
    dis = jax.lax.rsqrt(deg_ref[...])
    g = jnp.concatenate([p_ref[0], p_ref[1]], axis=1) * dis + b_ref[...]
    g = jnp.where(g >= 0, g, NEG_SLOPE * g)
    u = jnp.dot(g, Wh_ref[0], preferred_element_type=jnp.float32)
    o_ref[0] = u * dis


def _stage_c_body(p_ref, deg_ref, b_ref, o_ref):
    dis = jax.lax.rsqrt(deg_ref[...])
    o_ref[...] = jnp.concatenate([p_ref[0], p_ref[1]], axis=1) * dis + b_ref[...]


_x_spec = pl.BlockSpec((ROW_BLK, D), lambda i, j: (i, 0))
_p_spec = pl.BlockSpec((NC, ROW_BLK, DH), lambda i, j: (0, i, 0))
_deg_spec = pl.BlockSpec((ROW_BLK, 1), lambda i, j: (i, 0))
_wfull_spec = pl.BlockSpec((D, D), lambda i, j: (0, 0))
_whalf_spec = pl.BlockSpec((1, D, DH), lambda i, j: (j, 0, 0))
_bfull_spec = pl.BlockSpec((1, D), lambda i, j: (0, 0))
_osplit_spec = pl.BlockSpec((1, ROW_BLK, DH), lambda i, j: (j, i, 0))
_osplit_sds = jax.ShapeDtypeStruct((NC, N_NODES, DH), jnp.float32)
_grid = (N_NODES // ROW_BLK, NC)


def _split_w(W):
    return W.reshape(D, NC, DH).transpose(1, 0, 2)


def _stage_a(x, deg2, W_in, b_in, W1):
    return pl.pallas_call(
        _stage_a_body, grid=_grid,
        in_specs=[_x_spec, _deg_spec, _wfull_spec, _bfull_spec, _whalf_spec],
        out_specs=_osplit_spec, out_shape=_osplit_sds,
    )(x, deg2, W_in, b_in[None, :], _split_w(W1))


def _stage_mid(p, deg2, b, W):
    return pl.pallas_call(
        _stage_mid_body, grid=_grid,
        in_specs=[_p_spec, _deg_spec, _bfull_spec, _whalf_spec],
        out_specs=_osplit_spec, out_shape=_osplit_sds,
    )(p, deg2, b[None, :], _split_w(W))


def _stage_c(p, deg2, b):
    return pl.pallas_call(
        _stage_c_body, grid=(N_NODES // ROW_BLK,),
        in_specs=[pl.BlockSpec((NC, ROW_BLK, DH), lambda i: (0, i, 0)),
                  pl.BlockSpec((ROW_BLK, 1), lambda i: (i, 0)),
                  pl.BlockSpec((1, D), lambda i: (0, 0))],
        out_specs=pl.BlockSpec((ROW_BLK, D), lambda i: (i, 0)),
        out_shape=jax.ShapeDtypeStruct((N_NODES, D), jnp.float32),
    )(p, deg2, b[None, :])


def kernel(x, edge_index, W_in, b_in, W1, b1, W2, b2, W3, b3):
    src = edge_index[0].astype(jnp.int32)
    dst = edge_index[1].astype(jnp.int32)
    loop = jnp.arange(N_NODES, dtype=jnp.int32)

    src_p = jnp.concatenate(
        [src, loop, jnp.zeros((E_SCAT - E_REAL,), jnp.int32)])
    dst_p = jnp.concatenate(
        [dst, loop, jnp.full((E_SCAT - E_REAL,), JUNK, jnp.int32)])
    src_w = jnp.pad(src_p.reshape(NS, NBLK, GB, CHUNK),
                    ((0, 0), (0, NBLK_TOT - NBLK), (0, 0), (0, 0)))
    dst_w = jnp.pad(dst_p.reshape(NS, NBLK, GB, CHUNK),
                    ((0, 0), (0, NBLK_TOT - NBLK), (0, 0), (0, 0)),
                    constant_values=JUNK)

    pdeg = _sc_degree(dst_w)
    deg2 = pdeg[0, :N_NODES, 0:1]

    hs = _stage_a(x, deg2, W_in, b_in, W1)
    p = _sc_segsum(hs, src_w, dst_w)
    hs = _stage_mid(p, deg2, b1, W2)
    p = _sc_segsum(hs, src_w, dst_w)
    hs = _stage_mid(p, deg2, b2, W3)
    p = _sc_segsum(hs, src_w, dst_w)
    return _stage_c(p, deg2, b3)


# speedup vs baseline: 20.7722x; 1.0223x over previous
import functools

import jax
import jax.numpy as jnp
from jax import lax
from jax.experimental import pallas as pl
from jax.experimental.pallas import tpu as pltpu
from jax.experimental.pallas import tpu_sc as plsc

N_NODES = 10000
D = 128
DH = D // 2
NEG_SLOPE = 0.01
ROW_BLK = 1000

NC, NS = 2, 16
CHUNK = 128
GB = 8
NBLK = 21
CPW = NBLK * GB
NBLK_TOT = NBLK + 2
NBUF = 4
LEAD = 1
SDEPTH = NBUF - LEAD
E_REAL = 320000 + N_NODES
E_SCAT = NS * CPW * CHUNK
E_PAD = NS * NBLK_TOT * GB * CHUNK
N_PAD = 10240
JUNK = N_NODES
ZROWS = N_PAD // NS
TROWS = N_NODES // NS

_sc_mesh = plsc.VectorSubcoreMesh(
    core_axis_name="c", subcore_axis_name="s", num_cores=NC, num_subcores=NS)


@functools.partial(
    pl.kernel,
    out_type=jax.ShapeDtypeStruct((NC, N_PAD, DH), jnp.float32),
    mesh=_sc_mesh,
    compiler_params=pltpu.CompilerParams(use_tc_tiling_on_sc=False),
    scratch_types=[
        pltpu.VMEM((3, GB, CHUNK), jnp.int32),
        pltpu.VMEM((3, GB, CHUNK), jnp.int32),
        [pltpu.VMEM((CHUNK, DH), jnp.float32)] * NBUF,
        pltpu.VMEM_SHARED((N_NODES, DH), jnp.float32),
        pltpu.VMEM_SHARED((N_PAD, DH), jnp.float32),
        [pltpu.SemaphoreType.DMA] * NBUF,
        [pltpu.SemaphoreType.DMA] * NBUF,
        [pltpu.SemaphoreType.DMA] * 3,
    ],
)
def _sc_segsum(hs_hbm, src_hbm, dst_hbm, out_hbm,
               src_i, dst_i, bufs, table, acc, gsems, ssems, isems):
    cid = lax.axis_index("c")
    sid = lax.axis_index("s")

    def _zrow(j, _):
        for k in range(DH // 16):
            bufs[0][j, pl.ds(k * 16, 16)] = jnp.zeros((16,), jnp.float32)
        return 0
    lax.fori_loop(0, CHUNK, _zrow, 0)

    def _zcopy(k, _):
        pltpu.sync_copy(bufs[0],
                        acc.at[pl.ds(sid * ZROWS + k * CHUNK, CHUNK)])
        return 0
    lax.fori_loop(0, ZROWS // CHUNK, _zcopy, 0)

    pltpu.sync_copy(hs_hbm.at[cid, pl.ds(sid * TROWS, TROWS)],
                    table.at[pl.ds(sid * TROWS, TROWS)])

    pltpu.sync_copy(src_hbm.at[sid, 0], src_i.at[0])
    pltpu.sync_copy(dst_hbm.at[sid, 0], dst_i.at[0])

    def _idx_load(j, q):
        pltpu.async_copy(src_hbm.at[sid, j], src_i.at[q], isems[q])
        pltpu.async_copy(dst_hbm.at[sid, j], dst_i.at[q], isems[q])

    def _idx_wait(q):
        pltpu.make_async_copy(src_hbm.at[sid, 0], src_i.at[q],
                              isems[q]).wait()
        pltpu.make_async_copy(dst_hbm.at[sid, 0], dst_i.at[q],
                              isems[q]).wait()

    _idx_load(1, 1)

    plsc.subcore_barrier()

    def _gather(q, r, b):
        pltpu.async_copy(table.at[src_i.at[q, r]], bufs[b], gsems[b])

    def _gwait(b):
        pltpu.make_async_copy(table.at[src_i.at[0, 0]], bufs[b],
                              gsems[b]).wait()

    def _scatter(q, r, b):
        pltpu.async_copy(bufs[b], acc.at[dst_i.at[q, r]], ssems[b],
                         add=True)

    def _swait(b):
        pltpu.make_async_copy(bufs[0], acc.at[dst_i.at[0, 0]],
                              ssems[b]).wait()

    def _block(j, q, first=False):
        qn = (q + 1) % 3
        for k in range(GB):
            b = k % NBUF
            bn = (k + LEAD) % NBUF
            if k == 2:
                _idx_load(j + 2, (q + 2) % 3)
            if k == GB - LEAD:
                _idx_wait(qn)
            if not (first and k < SDEPTH):
                _swait(bn)
            if k + LEAD < GB:
                _gather(q, k + LEAD, bn)
            else:
                _gather(qn, k + LEAD - GB, bn)
            _gwait(b)
            _scatter(q, k, b)

    for b in range(LEAD):
        _gather(0, b, b)

    _block(0, 0, first=True)

    def _step3(i, _):
        j0 = 3 * i + 1
        _block(j0, 1)
        _block(j0 + 1, 2)
        _block(j0 + 2, 0)
        return 0
    lax.fori_loop(0, (NBLK - 3) // 3, _step3, 0)
    _block(NBLK - 2, (NBLK - 2) % 3)
    _block(NBLK - 1, (NBLK - 1) % 3)

    for k in range(SDEPTH):
        _swait((GB - SDEPTH + k) % NBUF)
    for k in range(LEAD):
        _gwait(k % NBUF)
    _idx_wait((NBLK + 1) % 3)

    plsc.subcore_barrier()

    pltpu.sync_copy(acc.at[pl.ds(sid * ZROWS, ZROWS)],
                    out_hbm.at[cid, pl.ds(sid * ZROWS, ZROWS)])


@functools.partial(
    pl.kernel,
    out_type=jax.ShapeDtypeStruct((NC, N_PAD, 16), jnp.float32),
    mesh=_sc_mesh,
    scratch_types=[
        pltpu.VMEM((NBLK_TOT, GB, CHUNK), jnp.int32),
        pltpu.VMEM((CHUNK, 16), jnp.float32),
        pltpu.VMEM_SHARED((N_PAD, 16), jnp.float32),
    ],
)
def _sc_degree(dst_hbm, out_hbm, dst_all, ones_v, acc):
    cid = lax.axis_index("c")
    sid = lax.axis_index("s")

    def _fill(val):
        def _f(j, _):
            ones_v[j, pl.ds(0, 16)] = jnp.full((16,), val, jnp.float32)
            return 0
        lax.fori_loop(0, CHUNK, _f, 0)

    _fill(0.0)

    def _zcopy(k, _):
        pltpu.sync_copy(ones_v,
                        acc.at[pl.ds(sid * ZROWS + k * CHUNK, CHUNK)])
        return 0
    lax.fori_loop(0, ZROWS // CHUNK, _zcopy, 0)

    pltpu.sync_copy(dst_hbm.at[sid], dst_all)
    _fill(1.0)

    plsc.subcore_barrier()

    def _step(j, _):
        for r in range(GB):
            pltpu.sync_copy(ones_v, acc.at[dst_all.at[j, r]], add=True)
        return 0
    half = (NBLK + 1) // 2
    lax.fori_loop(cid * half, jnp.minimum((cid + 1) * half, NBLK), _step, 0)

    plsc.subcore_barrier()

    pltpu.sync_copy(acc.at[pl.ds(sid * ZROWS, ZROWS)],
                    out_hbm.at[cid, pl.ds(sid * ZROWS, ZROWS)])


def _stage_a1_body(x_ref, Win_ref, bin_ref, W1h_ref, o_ref):
    t = jnp.dot(x_ref[...], Win_ref[...],
                preferred_element_type=jnp.float32) + bin_ref[...]
    o_ref[0] = jnp.dot(t, W1h_ref[0], preferred_element_type=jnp.float32)


def _stage_a2_body(u_ref, deg_ref, o_ref):
    dis = jax.lax.rsqrt(deg_ref[...])
    o_ref[0] = u_ref[0] * dis


def _stage_mid_body(p_ref, deg_ref, b_ref, Wh_ref, o_ref):
    dis = jax.lax.rsqrt(deg_ref[...])
    g = jnp.concatenate([p_ref[0], p_ref[1]], axis=1) * dis + b_ref[...]
    g = jnp.where(g >= 0, g, NEG_SLOPE * g)
    u = jnp.dot(g, Wh_ref[0], preferred_element_type=jnp.float32)
    o_ref[0] = u * dis


def _stage_c_body(p_ref, deg_ref, b_ref, o_ref):
    dis = jax.lax.rsqrt(deg_ref[...])
    o_ref[...] = jnp.concatenate([p_ref[0], p_ref[1]], axis=1) * dis + b_ref[...]


_x_spec = pl.BlockSpec((ROW_BLK, D), lambda i, j: (i, 0))
_p_spec = pl.BlockSpec((NC, ROW_BLK, DH), lambda i, j: (0, i, 0))
_deg_spec = pl.BlockSpec((ROW_BLK, 1), lambda i, j: (i, 0))
_wfull_spec = pl.BlockSpec((D, D), lambda i, j: (0, 0))
_whalf_spec = pl.BlockSpec((1, D, DH), lambda i, j: (j, 0, 0))
_bfull_spec = pl.BlockSpec((1, D), lambda i, j: (0, 0))
_osplit_spec = pl.BlockSpec((1, ROW_BLK, DH), lambda i, j: (j, i, 0))
_osplit_sds = jax.ShapeDtypeStruct((NC, N_NODES, DH), jnp.float32)
_grid = (N_NODES // ROW_BLK, NC)


def _split_w(W):
    return W.reshape(D, NC, DH).transpose(1, 0, 2)


def _stage_a1(x, W_in, b_in, W1):
    return pl.pallas_call(
        _stage_a1_body, grid=_grid,
        in_specs=[_x_spec, _wfull_spec, _bfull_spec, _whalf_spec],
        out_specs=_osplit_spec, out_shape=_osplit_sds,
    )(x, W_in, b_in[None, :], _split_w(W1))


def _stage_a2(u, deg2):
    return pl.pallas_call(
        _stage_a2_body, grid=_grid,
        in_specs=[pl.BlockSpec((1, ROW_BLK, DH), lambda i, j: (j, i, 0)),
                  _deg_spec],
        out_specs=_osplit_spec, out_shape=_osplit_sds,
    )(u, deg2)


def _stage_mid(p, deg2, b, W):
    return pl.pallas_call(
        _stage_mid_body, grid=_grid,
        in_specs=[_p_spec, _deg_spec, _bfull_spec, _whalf_spec],
        out_specs=_osplit_spec, out_shape=_osplit_sds,
    )(p, deg2, b[None, :], _split_w(W))


def _stage_c(p, deg2, b):
    return pl.pallas_call(
        _stage_c_body, grid=(N_NODES // ROW_BLK,),
        in_specs=[pl.BlockSpec((NC, ROW_BLK, DH), lambda i: (0, i, 0)),
                  pl.BlockSpec((ROW_BLK, 1), lambda i: (i, 0)),
                  pl.BlockSpec((1, D), lambda i: (0, 0))],
        out_specs=pl.BlockSpec((ROW_BLK, D), lambda i: (i, 0)),
        out_shape=jax.ShapeDtypeStruct((N_NODES, D), jnp.float32),
    )(p, deg2, b[None, :])


def kernel(x, edge_index, W_in, b_in, W1, b1, W2, b2, W3, b3):
    src = edge_index[0].astype(jnp.int32)
    dst = edge_index[1].astype(jnp.int32)
    loop = jnp.arange(N_NODES, dtype=jnp.int32)

    src_p = jnp.concatenate(
        [src, loop, jnp.zeros((E_SCAT - E_REAL,), jnp.int32)])
    dst_p = jnp.concatenate(
        [dst, loop, jnp.full((E_SCAT - E_REAL,), JUNK, jnp.int32)])
    src_w = jnp.pad(src_p.reshape(NS, NBLK, GB, CHUNK),
                    ((0, 0), (0, NBLK_TOT - NBLK), (0, 0), (0, 0)))
    dst_w = jnp.pad(dst_p.reshape(NS, NBLK, GB, CHUNK),
                    ((0, 0), (0, NBLK_TOT - NBLK), (0, 0), (0, 0)),
                    constant_values=JUNK)

    u1 = _stage_a1(x, W_in, b_in, W1)
    pdeg = _sc_degree(dst_w)
    deg2 = pdeg[0, :N_NODES, 0:1] + pdeg[1, :N_NODES, 0:1]

    hs = _stage_a2(u1, deg2)
    p = _sc_segsum(hs, src_w, dst_w)
    hs = _stage_mid(p, deg2, b1, W2)
    p = _sc_segsum(hs, src_w, dst_w)
    hs = _stage_mid(p, deg2, b2, W3)
    p = _sc_segsum(hs, src_w, dst_w)
    return _stage_c(p, deg2, b3)


# speedup vs baseline: 24.5208x; 1.1805x over previous
import functools

import jax
import jax.numpy as jnp
from jax import lax
from jax.experimental import pallas as pl
from jax.experimental.pallas import tpu as pltpu
from jax.experimental.pallas import tpu_sc as plsc

N_NODES = 10000
D = 128
DH = D // 2
NEG_SLOPE = 0.01
ROW_BLK = 1000

NC, NS = 2, 16
CHUNK = 128
GB = 8
NBLK = 21
CPW = NBLK * GB
NBLK_TOT = NBLK + 2
NBUF = 4
LEAD = 1
SDEPTH = NBUF - LEAD
E_REAL = 320000 + N_NODES
E_SCAT = NS * CPW * CHUNK
N_PAD = 10240
JUNK = N_NODES
ZROWS = N_PAD // NS
TROWS = 1000

_sc_mesh = plsc.VectorSubcoreMesh(
    core_axis_name="c", subcore_axis_name="s", num_cores=NC, num_subcores=NS)


@functools.partial(
    pl.kernel,
    out_type=jax.ShapeDtypeStruct((N_PAD, D), jnp.float32),
    mesh=_sc_mesh,
    compiler_params=pltpu.CompilerParams(use_tc_tiling_on_sc=False),
    scratch_types=[
        pltpu.VMEM((3, GB, CHUNK), jnp.int32),
        pltpu.VMEM((3, GB, CHUNK), jnp.int32),
        [pltpu.VMEM((CHUNK, DH), jnp.float32)] * NBUF,
        pltpu.VMEM_SHARED((N_NODES, DH), jnp.float32),
        pltpu.VMEM_SHARED((N_PAD, DH), jnp.float32),
        [pltpu.SemaphoreType.DMA] * NBUF,
        [pltpu.SemaphoreType.DMA] * NBUF,
        [pltpu.SemaphoreType.DMA] * 3,
    ],
)
def _sc_segsum(hs_hbm, src_hbm, dst_hbm, out_hbm,
               src_i, dst_i, bufs, table, acc, gsems, ssems, isems):
    cid = lax.axis_index("c")
    sid = lax.axis_index("s")

    def _zrow(j, _):
        for k in range(DH // 16):
            bufs[0][j, pl.ds(k * 16, 16)] = jnp.zeros((16,), jnp.float32)
        return 0
    lax.fori_loop(0, CHUNK, _zrow, 0)

    def _zcopy(k, _):
        pltpu.sync_copy(bufs[0],
                        acc.at[pl.ds(sid * ZROWS + k * CHUNK, CHUNK)])
        return 0
    lax.fori_loop(0, ZROWS // CHUNK, _zcopy, 0)

    @pl.when(sid < 10)
    def _stage_table():
        pltpu.sync_copy(
            hs_hbm.at[pl.ds(sid * TROWS, TROWS), pl.ds(cid * DH, DH)],
            table.at[pl.ds(sid * TROWS, TROWS)])

    pltpu.sync_copy(src_hbm.at[sid, 0], src_i.at[0])
    pltpu.sync_copy(dst_hbm.at[sid, 0], dst_i.at[0])

    def _idx_load(j, q):
        pltpu.async_copy(src_hbm.at[sid, j], src_i.at[q], isems[q])
        pltpu.async_copy(dst_hbm.at[sid, j], dst_i.at[q], isems[q])

    def _idx_wait(q):
        pltpu.make_async_copy(src_hbm.at[sid, 0], src_i.at[q],
                              isems[q]).wait()
        pltpu.make_async_copy(dst_hbm.at[sid, 0], dst_i.at[q],
                              isems[q]).wait()

    _idx_load(1, 1)

    plsc.subcore_barrier()

    def _gather(q, r, b):
        pltpu.async_copy(table.at[src_i.at[q, r]], bufs[b], gsems[b])

    def _gwait(b):
        pltpu.make_async_copy(table.at[src_i.at[0, 0]], bufs[b],
                              gsems[b]).wait()

    def _scatter(q, r, b):
        pltpu.async_copy(bufs[b], acc.at[dst_i.at[q, r]], ssems[b],
                         add=True)

    def _swait(b):
        pltpu.make_async_copy(bufs[0], acc.at[dst_i.at[0, 0]],
                              ssems[b]).wait()

    def _block(j, q, first=False):
        qn = (q + 1) % 3
        for k in range(GB):
            b = k % NBUF
            bn = (k + LEAD) % NBUF
            if k == 2:
                _idx_load(j + 2, (q + 2) % 3)
            if k == GB - LEAD:
                _idx_wait(qn)
            if not (first and k < SDEPTH):
                _swait(bn)
            if k + LEAD < GB:
                _gather(q, k + LEAD, bn)
            else:
                _gather(qn, k + LEAD - GB, bn)
            _gwait(b)
            _scatter(q, k, b)

    for b in range(LEAD):
        _gather(0, b, b)

    _block(0, 0, first=True)

    def _step3(i, _):
        j0 = 3 * i + 1
        _block(j0, 1)
        _block(j0 + 1, 2)
        _block(j0 + 2, 0)
        return 0
    lax.fori_loop(0, (NBLK - 3) // 3, _step3, 0)
    _block(NBLK - 2, (NBLK - 2) % 3)
    _block(NBLK - 1, (NBLK - 1) % 3)

    for k in range(SDEPTH):
        _swait((GB - SDEPTH + k) % NBUF)
    for k in range(LEAD):
        _gwait(k % NBUF)
    _idx_wait((NBLK + 1) % 3)

    plsc.subcore_barrier()

    pltpu.sync_copy(acc.at[pl.ds(sid * ZROWS, ZROWS)],
                    out_hbm.at[pl.ds(sid * ZROWS, ZROWS),
                               pl.ds(cid * DH, DH)])


@functools.partial(
    pl.kernel,
    out_type=jax.ShapeDtypeStruct((NC, N_PAD, 16), jnp.float32),
    mesh=_sc_mesh,
    scratch_types=[
        pltpu.VMEM((NBLK_TOT, GB, CHUNK), jnp.int32),
        pltpu.VMEM((CHUNK, 16), jnp.float32),
        pltpu.VMEM_SHARED((N_PAD, 16), jnp.float32),
    ],
)
def _sc_degree(dst_hbm, out_hbm, dst_all, ones_v, acc):
    cid = lax.axis_index("c")
    sid = lax.axis_index("s")

    def _fill(val):
        def _f(j, _):
            ones_v[j, pl.ds(0, 16)] = jnp.full((16,), val, jnp.float32)
            return 0
        lax.fori_loop(0, CHUNK, _f, 0)

    _fill(0.0)

    def _zcopy(k, _):
        pltpu.sync_copy(ones_v,
                        acc.at[pl.ds(sid * ZROWS + k * CHUNK, CHUNK)])
        return 0
    lax.fori_loop(0, ZROWS // CHUNK, _zcopy, 0)

    pltpu.sync_copy(dst_hbm.at[sid], dst_all)
    _fill(1.0)

    plsc.subcore_barrier()

    def _step(j, _):
        for r in range(GB):
            pltpu.sync_copy(ones_v, acc.at[dst_all.at[j, r]], add=True)
        return 0
    half = (NBLK + 1) // 2
    lax.fori_loop(cid * half, jnp.minimum((cid + 1) * half, NBLK), _step, 0)

    plsc.subcore_barrier()

    pltpu.sync_copy(acc.at[pl.ds(sid * ZROWS, ZROWS)],
                    out_hbm.at[cid, pl.ds(sid * ZROWS, ZROWS)])


def _stage_a1_body(x_ref, Win_ref, bin_ref, W1_ref, o_ref):
    t = jnp.dot(x_ref[...], Win_ref[...],
                preferred_element_type=jnp.float32) + bin_ref[...]
    o_ref[...] = jnp.dot(t, W1_ref[...], preferred_element_type=jnp.float32)


def _stage_a2_body(u_ref, deg_ref, o_ref):
    dis = jax.lax.rsqrt(deg_ref[...])
    o_ref[...] = u_ref[...] * dis


def _stage_mid_body(p_ref, deg_ref, b_ref, W_ref, o_ref):
    dis = jax.lax.rsqrt(deg_ref[...])
    g = p_ref[...] * dis + b_ref[...]
    g = jnp.where(g >= 0, g, NEG_SLOPE * g)
    u = jnp.dot(g, W_ref[...], preferred_element_type=jnp.float32)
    o_ref[...] = u * dis


def _stage_c_body(p_ref, deg_ref, b_ref, o_ref):
    dis = jax.lax.rsqrt(deg_ref[...])
    o_ref[...] = p_ref[...] * dis + b_ref[...]


_row_spec = pl.BlockSpec((ROW_BLK, D), lambda i: (i, 0))
_deg_spec = pl.BlockSpec((ROW_BLK, 1), lambda i: (i, 0))
_w_spec = pl.BlockSpec((D, D), lambda i: (0, 0))
_b_spec = pl.BlockSpec((1, D), lambda i: (0, 0))
_out_sds = jax.ShapeDtypeStruct((N_NODES, D), jnp.float32)
_grid = (N_NODES // ROW_BLK,)


def _stage_a1(x, W_in, b_in, W1):
    return pl.pallas_call(
        _stage_a1_body, grid=_grid,
        in_specs=[_row_spec, _w_spec, _b_spec, _w_spec],
        out_specs=_row_spec, out_shape=_out_sds,
    )(x, W_in, b_in[None, :], W1)


def _stage_a2(u, deg2):
    return pl.pallas_call(
        _stage_a2_body, grid=_grid,
        in_specs=[_row_spec, _deg_spec],
        out_specs=_row_spec, out_shape=_out_sds,
    )(u, deg2)


def _stage_mid(p, deg2, b, W):
    return pl.pallas_call(
        _stage_mid_body, grid=_grid,
        in_specs=[_row_spec, _deg_spec, _b_spec, _w_spec],
        out_specs=_row_spec, out_shape=_out_sds,
    )(p, deg2, b[None, :], W)


def _stage_c(p, deg2, b):
    return pl.pallas_call(
        _stage_c_body, grid=_grid,
        in_specs=[_row_spec, _deg_spec, _b_spec],
        out_specs=_row_spec, out_shape=_out_sds,
    )(p, deg2, b[None, :])


def kernel(x, edge_index, W_in, b_in, W1, b1, W2, b2, W3, b3):
    src = edge_index[0].astype(jnp.int32)
    dst = edge_index[1].astype(jnp.int32)
    loop = jnp.arange(N_NODES, dtype=jnp.int32)

    src_p = jnp.concatenate(
        [src, loop, jnp.zeros((E_SCAT - E_REAL,), jnp.int32)])
    dst_p = jnp.concatenate(
        [dst, loop, jnp.full((E_SCAT - E_REAL,), JUNK, jnp.int32)])
    src_w = jnp.pad(src_p.reshape(NS, NBLK, GB, CHUNK),
                    ((0, 0), (0, NBLK_TOT - NBLK), (0, 0), (0, 0)))
    dst_w = jnp.pad(dst_p.reshape(NS, NBLK, GB, CHUNK),
                    ((0, 0), (0, NBLK_TOT - NBLK), (0, 0), (0, 0)),
                    constant_values=JUNK)

    u1 = _stage_a1(x, W_in, b_in, W1)
    pdeg = _sc_degree(dst_w)
    deg2 = pdeg[0, :N_NODES, 0:1] + pdeg[1, :N_NODES, 0:1]

    hs = _stage_a2(u1, deg2)
    p = _sc_segsum(hs, src_w, dst_w)
    hs = _stage_mid(p, deg2, b1, W2)
    p = _sc_segsum(hs, src_w, dst_w)
    hs = _stage_mid(p, deg2, b2, W3)
    p = _sc_segsum(hs, src_w, dst_w)
    return _stage_c(p, deg2, b3)


# speedup vs baseline: 25.8074x; 1.0525x over previous
import functools

import jax
import jax.numpy as jnp
from jax import lax
from jax.experimental import pallas as pl
from jax.experimental.pallas import tpu as pltpu
from jax.experimental.pallas import tpu_sc as plsc

N_NODES = 10000
D = 128
DH = D // 2
NEG_SLOPE = 0.01
ROW_BLK = 1000

NC, NS = 2, 16
CHUNK = 128
GB = 8
NBLK = 20
CPW = NBLK * GB
NBLK_TOT = NBLK + 2
NBUF = 4
LEAD = 1
SDEPTH = NBUF - LEAD
E_REAL = 320000
E_SCAT = NS * CPW * CHUNK
N_PAD = 10240
JUNK = N_NODES
ZROWS = N_PAD // NS
TROWS = 1000

_sc_mesh = plsc.VectorSubcoreMesh(
    core_axis_name="c", subcore_axis_name="s", num_cores=NC, num_subcores=NS)


@functools.partial(
    pl.kernel,
    out_type=jax.ShapeDtypeStruct((N_PAD, D), jnp.float32),
    mesh=_sc_mesh,
    compiler_params=pltpu.CompilerParams(use_tc_tiling_on_sc=False),
    scratch_types=[
        pltpu.VMEM((3, GB, CHUNK), jnp.int32),
        pltpu.VMEM((3, GB, CHUNK), jnp.int32),
        [pltpu.VMEM((CHUNK, DH), jnp.float32)] * NBUF,
        pltpu.VMEM_SHARED((N_NODES, DH), jnp.float32),
        pltpu.VMEM_SHARED((N_PAD, DH), jnp.float32),
        [pltpu.SemaphoreType.DMA] * NBUF,
        [pltpu.SemaphoreType.DMA] * NBUF,
        [pltpu.SemaphoreType.DMA] * 3,
    ],
)
def _sc_segsum(hs_hbm, src_hbm, dst_hbm, out_hbm,
               src_i, dst_i, bufs, table, acc, gsems, ssems, isems):
    cid = lax.axis_index("c")
    sid = lax.axis_index("s")

    def _zrow(j, _):
        for k in range(DH // 16):
            bufs[0][j, pl.ds(k * 16, 16)] = jnp.zeros((16,), jnp.float32)
        return 0
    lax.fori_loop(0, CHUNK, _zrow, 0)

    def _zcopy(k, _):
        pltpu.sync_copy(bufs[0],
                        acc.at[pl.ds(sid * ZROWS + k * CHUNK, CHUNK)])
        return 0
    lax.fori_loop(0, ZROWS // CHUNK, _zcopy, 0)

    @pl.when(sid < 10)
    def _stage_table():
        pltpu.sync_copy(
            hs_hbm.at[pl.ds(sid * TROWS, TROWS), pl.ds(cid * DH, DH)],
            table.at[pl.ds(sid * TROWS, TROWS)])

    pltpu.sync_copy(src_hbm.at[sid, 0], src_i.at[0])
    pltpu.sync_copy(dst_hbm.at[sid, 0], dst_i.at[0])

    def _idx_load(j, q):
        pltpu.async_copy(src_hbm.at[sid, j], src_i.at[q], isems[q])
        pltpu.async_copy(dst_hbm.at[sid, j], dst_i.at[q], isems[q])

    def _idx_wait(q):
        pltpu.make_async_copy(src_hbm.at[sid, 0], src_i.at[q],
                              isems[q]).wait()
        pltpu.make_async_copy(dst_hbm.at[sid, 0], dst_i.at[q],
                              isems[q]).wait()

    _idx_load(1, 1)

    plsc.subcore_barrier()

    def _gather(q, r, b):
        pltpu.async_copy(table.at[src_i.at[q, r]], bufs[b], gsems[b])

    def _gwait(b):
        pltpu.make_async_copy(table.at[src_i.at[0, 0]], bufs[b],
                              gsems[b]).wait()

    def _scatter(q, r, b):
        pltpu.async_copy(bufs[b], acc.at[dst_i.at[q, r]], ssems[b],
                         add=True)

    def _swait(b):
        pltpu.make_async_copy(bufs[0], acc.at[dst_i.at[0, 0]],
                              ssems[b]).wait()

    def _block(j, q, first=False):
        qn = (q + 1) % 3
        for k in range(GB):
            b = k % NBUF
            bn = (k + LEAD) % NBUF
            if k == 2:
                _idx_load(j + 2, (q + 2) % 3)
            if k == GB - LEAD:
                _idx_wait(qn)
            if not (first and k < SDEPTH):
                _swait(bn)
            if k + LEAD < GB:
                _gather(q, k + LEAD, bn)
            else:
                _gather(qn, k + LEAD - GB, bn)
            _gwait(b)
            _scatter(q, k, b)

    for b in range(LEAD):
        _gather(0, b, b)

    _block(0, 0, first=True)

    def _step3(i, _):
        j0 = 3 * i + 1
        _block(j0, 1)
        _block(j0 + 1, 2)
        _block(j0 + 2, 0)
        return 0
    _M3 = (NBLK - 4) // 3
    lax.fori_loop(0, _M3, _step3, 0)
    for j in range(3 * _M3 + 1, NBLK):
        _block(j, j % 3)

    for k in range(SDEPTH):
        _swait((GB - SDEPTH + k) % NBUF)
    for k in range(LEAD):
        _gwait(k % NBUF)
    _idx_wait((NBLK + 1) % 3)

    plsc.subcore_barrier()

    pltpu.sync_copy(acc.at[pl.ds(sid * ZROWS, ZROWS)],
                    out_hbm.at[pl.ds(sid * ZROWS, ZROWS),
                               pl.ds(cid * DH, DH)])


@functools.partial(
    pl.kernel,
    out_type=jax.ShapeDtypeStruct((NC, N_PAD, 16), jnp.float32),
    mesh=_sc_mesh,
    scratch_types=[
        pltpu.VMEM((NBLK_TOT, GB, CHUNK), jnp.int32),
        pltpu.VMEM((CHUNK, 16), jnp.float32),
        pltpu.VMEM_SHARED((N_PAD, 16), jnp.float32),
    ],
)
def _sc_degree(dst_hbm, out_hbm, dst_all, ones_v, acc):
    cid = lax.axis_index("c")
    sid = lax.axis_index("s")

    def _fill(val):
        def _f(j, _):
            ones_v[j, pl.ds(0, 16)] = jnp.full((16,), val, jnp.float32)
            return 0
        lax.fori_loop(0, CHUNK, _f, 0)

    _fill(0.0)

    def _zcopy(k, _):
        pltpu.sync_copy(ones_v,
                        acc.at[pl.ds(sid * ZROWS + k * CHUNK, CHUNK)])
        return 0
    lax.fori_loop(0, ZROWS // CHUNK, _zcopy, 0)

    pltpu.sync_copy(dst_hbm.at[sid], dst_all)
    _fill(1.0)

    plsc.subcore_barrier()

    def _step(j, _):
        for r in range(GB):
            pltpu.sync_copy(ones_v, acc.at[dst_all.at[j, r]], add=True)
        return 0
    half = (NBLK + 1) // 2
    lax.fori_loop(cid * half, jnp.minimum((cid + 1) * half, NBLK), _step, 0)

    plsc.subcore_barrier()

    pltpu.sync_copy(acc.at[pl.ds(sid * ZROWS, ZROWS)],
                    out_hbm.at[cid, pl.ds(sid * ZROWS, ZROWS)])


def _stage_a1_body(x_ref, Win_ref, bin_ref, W1_ref, o_ref):
    t = jnp.dot(x_ref[...], Win_ref[...],
                preferred_element_type=jnp.float32) + bin_ref[...]
    o_ref[...] = jnp.dot(t, W1_ref[...], preferred_element_type=jnp.float32)


def _stage_a2_body(u_ref, deg_ref, o_ref):
    dis = jax.lax.rsqrt(deg_ref[...])
    o_ref[...] = u_ref[...] * dis


def _stage_mid_body(p_ref, hs_ref, deg_ref, b_ref, W_ref, o_ref):
    dis = jax.lax.rsqrt(deg_ref[...])
    g = (p_ref[...] + hs_ref[...]) * dis + b_ref[...]
    g = jnp.where(g >= 0, g, NEG_SLOPE * g)
    u = jnp.dot(g, W_ref[...], preferred_element_type=jnp.float32)
    o_ref[...] = u * dis


def _stage_c_body(p_ref, hs_ref, deg_ref, b_ref, o_ref):
    dis = jax.lax.rsqrt(deg_ref[...])
    o_ref[...] = (p_ref[...] + hs_ref[...]) * dis + b_ref[...]


_row_spec = pl.BlockSpec((ROW_BLK, D), lambda i: (i, 0))
_deg_spec = pl.BlockSpec((ROW_BLK, 1), lambda i: (i, 0))
_w_spec = pl.BlockSpec((D, D), lambda i: (0, 0))
_b_spec = pl.BlockSpec((1, D), lambda i: (0, 0))
_out_sds = jax.ShapeDtypeStruct((N_NODES, D), jnp.float32)
_grid = (N_NODES // ROW_BLK,)


def _stage_a1(x, W_in, b_in, W1):
    return pl.pallas_call(
        _stage_a1_body, grid=_grid,
        in_specs=[_row_spec, _w_spec, _b_spec, _w_spec],
        out_specs=_row_spec, out_shape=_out_sds,
    )(x, W_in, b_in[None, :], W1)


def _stage_a2(u, deg2):
    return pl.pallas_call(
        _stage_a2_body, grid=_grid,
        in_specs=[_row_spec, _deg_spec],
        out_specs=_row_spec, out_shape=_out_sds,
    )(u, deg2)


def _stage_mid(p, hs, deg2, b, W):
    return pl.pallas_call(
        _stage_mid_body, grid=_grid,
        in_specs=[_row_spec, _row_spec, _deg_spec, _b_spec, _w_spec],
        out_specs=_row_spec, out_shape=_out_sds,
    )(p, hs, deg2, b[None, :], W)


def _stage_c(p, hs, deg2, b):
    return pl.pallas_call(
        _stage_c_body, grid=_grid,
        in_specs=[_row_spec, _row_spec, _deg_spec, _b_spec],
        out_specs=_row_spec, out_shape=_out_sds,
    )(p, hs, deg2, b[None, :])


def kernel(x, edge_index, W_in, b_in, W1, b1, W2, b2, W3, b3):
    src = edge_index[0].astype(jnp.int32)
    dst = edge_index[1].astype(jnp.int32)

    src_p = jnp.concatenate([src, jnp.zeros((E_SCAT - E_REAL,), jnp.int32)])
    dst_p = jnp.concatenate([dst, jnp.full((E_SCAT - E_REAL,), JUNK,
                                           jnp.int32)])
    src_w = jnp.pad(src_p.reshape(NS, NBLK, GB, CHUNK),
                    ((0, 0), (0, NBLK_TOT - NBLK), (0, 0), (0, 0)))
    dst_w = jnp.pad(dst_p.reshape(NS, NBLK, GB, CHUNK),
                    ((0, 0), (0, NBLK_TOT - NBLK), (0, 0), (0, 0)),
                    constant_values=JUNK)

    u1 = _stage_a1(x, W_in, b_in, W1)
    pdeg = _sc_degree(dst_w)
    deg2 = pdeg[0, :N_NODES, 0:1] + pdeg[1, :N_NODES, 0:1] + 1.0

    hs = _stage_a2(u1, deg2)
    p = _sc_segsum(hs, src_w, dst_w)
    hs = _stage_mid(p, hs, deg2, b1, W2)
    p = _sc_segsum(hs, src_w, dst_w)
    hs = _stage_mid(p, hs, deg2, b2, W3)
    p = _sc_segsum(hs, src_w, dst_w)
    return _stage_c(p, hs, deg2, b3)
